# baseline (speedup 1.0000x reference)
import functools

import jax
import jax.numpy as jnp
from jax import lax
from jax.experimental import pallas as pl
from jax.experimental.pallas import tpu as pltpu
from jax.experimental.pallas import tpu_sc as plsc

N = 10000
E = 320000
D = 128
H = 128
G = 64
NC_OUT = 10

NCORES = 2
NSUB = 16
NW = NCORES * NSUB
CHUNK = 128
NSTEPS = 81
EPW = NSTEPS * CHUNK
EPAD = EPW * NW
NRING = 3
ACC_ROWS = 10112
ZROWS = ACC_ROWS // NSUB


@functools.partial(
    pl.kernel,
    out_type=jax.ShapeDtypeStruct((NCORES, ACC_ROWS, D), jnp.float32),
    mesh=plsc.VectorSubcoreMesh(core_axis_name="c", subcore_axis_name="s"),
    scratch_types=[
        [pltpu.VMEM((2, CHUNK), jnp.int32)] * NRING,
        [pltpu.VMEM((CHUNK, D), jnp.float32)] * NRING,
        pltpu.VMEM_SHARED((ACC_ROWS, D), jnp.float32),
        [pltpu.SemaphoreType.DMA] * NRING,
        [pltpu.SemaphoreType.DMA] * NRING,
    ],
)
def _sc_agg(x_hbm, idx_hbm, zeros_hbm, out_hbm,
            ibuf, rows, acc, isems, gsems):
    c = lax.axis_index("c")
    s = lax.axis_index("s")
    w = c * NSUB + s

    pltpu.sync_copy(zeros_hbm, acc.at[pl.ds(s * ZROWS, ZROWS)])

    for k in range(NRING):
        pltpu.async_copy(idx_hbm.at[w, k], ibuf[k], isems[k])
    plsc.subcore_barrier()
    for b in range(2):
        pltpu.make_async_copy(idx_hbm.at[w, b], ibuf[b], isems[b]).wait()
        pltpu.async_copy(x_hbm.at[ibuf[b].at[0]], rows[b], gsems[b])

    def outer(g, _):
        for bb in range(NRING):
            j = g * NRING + bb
            nb = (bb + 2) % NRING
            pltpu.make_async_copy(x_hbm.at[ibuf[bb].at[0]], rows[bb],
                                  gsems[bb]).wait()

            @pl.when(j + 2 < NSTEPS)
            def _():
                pltpu.make_async_copy(idx_hbm.at[w, j + 2], ibuf[nb],
                                      isems[nb]).wait()
                pltpu.async_copy(x_hbm.at[ibuf[nb].at[0]], rows[nb],
                                 gsems[nb])

            pltpu.sync_copy(rows[bb], acc.at[ibuf[bb].at[1]], add=True)

            @pl.when(j + NRING < NSTEPS)
            def _():
                pltpu.async_copy(idx_hbm.at[w, j + NRING], ibuf[bb],
                                 isems[bb])

        return 0

    lax.fori_loop(0, NSTEPS // NRING, outer, 0)
    plsc.subcore_barrier()

    for off, sz in ((0, 128), (128, 128), (256, 128), (384, 128), (512, 120)):
        r0 = s * ZROWS + off
        buf = rows[0].at[pl.ds(0, sz)]
        pltpu.sync_copy(acc.at[pl.ds(r0, sz)], buf)
        pltpu.sync_copy(buf, out_hbm.at[c, pl.ds(r0, sz)])


BN = 2000
NBLK = N // BN


def _mm(a, b):
    return jnp.dot(a, b, preferred_element_type=jnp.float32,
                   precision=lax.Precision.HIGHEST)


def _gin_math(x, p0, p1, w1_ref, b1_ref, w2_ref, b2_ref):
    h = x + p0 + p1
    h = _mm(h, w1_ref[...]) + b1_ref[...]
    h = jnp.maximum(h, 0.0)
    h = _mm(h, w2_ref[...]) + b2_ref[...]
    return jnp.maximum(h, 0.0)


def _tc_layer_body(x_ref, p0_ref, p1_ref, w1_ref, b1_ref, w2_ref, b2_ref,
                   o_ref):
    o_ref[...] = _gin_math(x_ref[...], p0_ref[0], p1_ref[0],
                           w1_ref, b1_ref, w2_ref, b2_ref)


_X_SPECS = [
    pl.BlockSpec((BN, D), lambda i: (i, 0)),
    pl.BlockSpec((1, BN, D), lambda i: (0, i, 0)),
    pl.BlockSpec((1, BN, D), lambda i: (1, i, 0)),
]
_W_SPECS = [
    pl.BlockSpec((D, H), lambda i: (0, 0)),
    pl.BlockSpec((1, H), lambda i: (0, 0)),
    pl.BlockSpec((H, H), lambda i: (0, 0)),
    pl.BlockSpec((1, H), lambda i: (0, 0)),
]


def _tc_layer(x, parts, w1, b1, w2, b2):
    return pl.pallas_call(
        _tc_layer_body,
        grid=(NBLK,),
        in_specs=_X_SPECS + _W_SPECS,
        out_specs=pl.BlockSpec((BN, H), lambda i: (i, 0)),
        out_shape=jax.ShapeDtypeStruct((N, H), jnp.float32),
    )(x, parts, parts, w1, b1, w2, b2)


def _tc_layer_pool_body(x_ref, p0_ref, p1_ref, w1_ref, b1_ref, w2_ref,
                        b2_ref, batch_ref, s_ref, c_ref):
    h = _gin_math(x_ref[...], p0_ref[0], p1_ref[0],
                  w1_ref, b1_ref, w2_ref, b2_ref)

    gids = lax.broadcasted_iota(jnp.int32, (BN, G), 1)
    onehot_t = (gids == batch_ref[...]).astype(jnp.float32)

    @pl.when(pl.program_id(0) == 0)
    def _():
        s_ref[...] = jnp.zeros_like(s_ref)
        c_ref[...] = jnp.zeros_like(c_ref)

    s_ref[...] += lax.dot_general(
        onehot_t, h, (((0,), (0,)), ((), ())),
        preferred_element_type=jnp.float32,
        precision=lax.Precision.HIGHEST)
    c_ref[...] += jnp.sum(onehot_t, axis=0)[None, :]


def _tc_layer_pool(x, parts, w1, b1, w2, b2, batch2):
    return pl.pallas_call(
        _tc_layer_pool_body,
        grid=(NBLK,),
        in_specs=_X_SPECS + _W_SPECS + [pl.BlockSpec((BN, 1), lambda i: (i, 0))],
        out_specs=[
            pl.BlockSpec((G, H), lambda i: (0, 0)),
            pl.BlockSpec((1, G), lambda i: (0, 0)),
        ],
        out_shape=[
            jax.ShapeDtypeStruct((G, H), jnp.float32),
            jax.ShapeDtypeStruct((1, G), jnp.float32),
        ],
    )(x, parts, parts, w1, b1, w2, b2, batch2)


def _tc_head_body(s0_ref, c0_ref, s1_ref, c1_ref, s2_ref, c2_ref,
                  mw_ref, mb_ref, fw1_ref, fb1_ref, fw2_ref, fb2_ref,
                  o_ref):
    def hg(s_ref, c_ref):
        cnt = jnp.maximum(c_ref[...], 1.0)
        pooled = s_ref[...] / cnt.reshape(G, 1)
        return _mm(pooled, mw_ref[...]) + mb_ref[...]

    acc = (_mm(hg(s0_ref, c0_ref), fw1_ref[0:H, :])
           + _mm(hg(s1_ref, c1_ref), fw1_ref[H:2 * H, :])
           + _mm(hg(s2_ref, c2_ref), fw1_ref[2 * H:3 * H, :]))
    acc = jnp.maximum(acc + fb1_ref[...], 0.0)
    o_ref[...] = _mm(acc, fw2_ref[...]) + fb2_ref[...]


def _tc_head(s0, c0, s1, c1, s2, c2, mlp_W, mlp_b2, fW1, fb1_2, fW2, fb2_2):
    return pl.pallas_call(
        _tc_head_body,
        out_shape=jax.ShapeDtypeStruct((G, NC_OUT), jnp.float32),
    )(s0, c0, s1, c1, s2, c2, mlp_W, mlp_b2, fW1, fb1_2, fW2, fb2_2)


def kernel(x_org, edge_index_org, batch_org, x_c1, edge_index_c1, batch_c1,
           x_c2, edge_index_c2, batch_c2, conv_W1, conv_b1, conv_W2, conv_b2,
           mlp_W, mlp_b, final_W1, final_b1, final_W2, final_b2):
    zeros = jnp.zeros((ZROWS, D), jnp.float32)
    pad_src = jnp.zeros((EPAD - E,), jnp.int32)
    pad_dst = jnp.full((EPAD - E,), ACC_ROWS - 1, jnp.int32)

    b1r = conv_b1.reshape(3, 1, H)
    b2r = conv_b2.reshape(3, 1, H)

    def branch(x, ei, batch):
        src = jnp.concatenate([ei[0], pad_src]).reshape(NW, NSTEPS, 1, CHUNK)
        dst = jnp.concatenate([ei[1], pad_dst]).reshape(NW, NSTEPS, 1, CHUNK)
        idx = jnp.concatenate([src, dst], axis=2)
        batch2 = batch.reshape(N, 1)
        for l in range(2):
            parts = _sc_agg(x, idx, zeros)
            x = _tc_layer(x, parts, conv_W1[l], b1r[l], conv_W2[l], b2r[l])
        parts = _sc_agg(x, idx, zeros)
        return _tc_layer_pool(x, parts, conv_W1[2], b1r[2], conv_W2[2],
                              b2r[2], batch2)

    s0, c0 = branch(x_org, edge_index_org, batch_org)
    s1, c1 = branch(x_c1, edge_index_c1, batch_c1)
    s2, c2 = branch(x_c2, edge_index_c2, batch_c2)

    return _tc_head(s0, c0, s1, c1, s2, c2,
                    mlp_W, mlp_b.reshape(1, H),
                    final_W1, final_b1.reshape(1, H),
                    final_W2, final_b2.reshape(1, NC_OUT))


# speedup vs baseline: 1.4050x; 1.4050x over previous
import functools

import jax
import jax.numpy as jnp
from jax import lax
from jax.experimental import pallas as pl
from jax.experimental.pallas import tpu as pltpu
from jax.experimental.pallas import tpu_sc as plsc

N = 10000
E = 320000
D = 128
H = 128
G = 64
NC_OUT = 10

NCORES = 2
NSUB = 16
NW = NCORES * NSUB
CHUNK = 128
HSTEPS = 40
NSTEPS = 2 * HSTEPS
EPW = NSTEPS * CHUNK
EPAD = EPW * NW
ACC_ROWS = 10112
ZROWS = ACC_ROWS // NSUB


@functools.partial(
    pl.kernel,
    out_type=jax.ShapeDtypeStruct((NCORES, ACC_ROWS, D), jnp.float32),
    mesh=plsc.VectorSubcoreMesh(core_axis_name="c", subcore_axis_name="s"),
    scratch_types=[
        pltpu.VMEM((HSTEPS, 2, CHUNK), jnp.int32),
        [pltpu.VMEM((CHUNK, D), jnp.float32)] * 2,
        pltpu.VMEM_SHARED((ACC_ROWS, D), jnp.float32),
        [pltpu.SemaphoreType.DMA] * 2,
    ],
)
def _sc_agg(x_hbm, idx_hbm, zeros_hbm, out_hbm, ibuf, rows, acc, gsems):
    c = lax.axis_index("c")
    s = lax.axis_index("s")
    w = c * NSUB + s

    def visit(j, b, issue):
        pltpu.make_async_copy(x_hbm.at[ibuf.at[j, 0]], rows[b],
                              gsems[b]).wait()
        pltpu.sync_copy(rows[b], acc.at[ibuf.at[j, 1]], add=True)
        if issue:
            pltpu.async_copy(x_hbm.at[ibuf.at[j + 2, 0]], rows[b], gsems[b])

    def run_half(h):
        for b in range(2):
            pltpu.async_copy(x_hbm.at[ibuf.at[b, 0]], rows[b], gsems[b])

        def outer(g, _):
            visit(2 * g, 0, True)
            visit(2 * g + 1, 1, True)
            return 0

        lax.fori_loop(0, HSTEPS // 2 - 1, outer, 0)
        visit(HSTEPS - 2, 0, False)
        visit(HSTEPS - 1, 1, False)

    pltpu.sync_copy(idx_hbm.at[w, 0], ibuf)
    pltpu.sync_copy(zeros_hbm, acc.at[pl.ds(s * ZROWS, ZROWS)])
    plsc.subcore_barrier()
    run_half(0)
    pltpu.sync_copy(idx_hbm.at[w, 1], ibuf)
    run_half(1)
    plsc.subcore_barrier()

    for off, sz in ((0, 128), (128, 128), (256, 128), (384, 128), (512, 120)):
        r0 = s * ZROWS + off
        buf = rows[0].at[pl.ds(0, sz)]
        pltpu.sync_copy(acc.at[pl.ds(r0, sz)], buf)
        pltpu.sync_copy(buf, out_hbm.at[c, pl.ds(r0, sz)])


BN = 2000
NBLK = N // BN


def _mm(a, b):
    return jnp.dot(a, b, preferred_element_type=jnp.float32,
                   precision=lax.Precision.HIGHEST)


def _gin_math(x, p0, p1, w1_ref, b1_ref, w2_ref, b2_ref):
    h = x + p0 + p1
    h = _mm(h, w1_ref[...]) + b1_ref[...]
    h = jnp.maximum(h, 0.0)
    h = _mm(h, w2_ref[...]) + b2_ref[...]
    return jnp.maximum(h, 0.0)


def _tc_layer_body(x_ref, p0_ref, p1_ref, w1_ref, b1_ref, w2_ref, b2_ref,
                   o_ref):
    o_ref[...] = _gin_math(x_ref[...], p0_ref[0], p1_ref[0],
                           w1_ref, b1_ref, w2_ref, b2_ref)


_X_SPECS = [
    pl.BlockSpec((BN, D), lambda i: (i, 0)),
    pl.BlockSpec((1, BN, D), lambda i: (0, i, 0)),
    pl.BlockSpec((1, BN, D), lambda i: (1, i, 0)),
]
_W_SPECS = [
    pl.BlockSpec((D, H), lambda i: (0, 0)),
    pl.BlockSpec((1, H), lambda i: (0, 0)),
    pl.BlockSpec((H, H), lambda i: (0, 0)),
    pl.BlockSpec((1, H), lambda i: (0, 0)),
]


def _tc_layer(x, parts, w1, b1, w2, b2):
    return pl.pallas_call(
        _tc_layer_body,
        grid=(NBLK,),
        in_specs=_X_SPECS + _W_SPECS,
        out_specs=pl.BlockSpec((BN, H), lambda i: (i, 0)),
        out_shape=jax.ShapeDtypeStruct((N, H), jnp.float32),
    )(x, parts, parts, w1, b1, w2, b2)


def _tc_layer_pool_body(x_ref, p0_ref, p1_ref, w1_ref, b1_ref, w2_ref,
                        b2_ref, batch_ref, s_ref, c_ref):
    h = _gin_math(x_ref[...], p0_ref[0], p1_ref[0],
                  w1_ref, b1_ref, w2_ref, b2_ref)

    gids = lax.broadcasted_iota(jnp.int32, (BN, G), 1)
    onehot_t = (gids == batch_ref[...]).astype(jnp.float32)

    @pl.when(pl.program_id(0) == 0)
    def _():
        s_ref[...] = jnp.zeros_like(s_ref)
        c_ref[...] = jnp.zeros_like(c_ref)

    s_ref[...] += lax.dot_general(
        onehot_t, h, (((0,), (0,)), ((), ())),
        preferred_element_type=jnp.float32,
        precision=lax.Precision.HIGHEST)
    c_ref[...] += jnp.sum(onehot_t, axis=0)[None, :]


def _tc_layer_pool(x, parts, w1, b1, w2, b2, batch2):
    return pl.pallas_call(
        _tc_layer_pool_body,
        grid=(NBLK,),
        in_specs=_X_SPECS + _W_SPECS + [pl.BlockSpec((BN, 1), lambda i: (i, 0))],
        out_specs=[
            pl.BlockSpec((G, H), lambda i: (0, 0)),
            pl.BlockSpec((1, G), lambda i: (0, 0)),
        ],
        out_shape=[
            jax.ShapeDtypeStruct((G, H), jnp.float32),
            jax.ShapeDtypeStruct((1, G), jnp.float32),
        ],
    )(x, parts, parts, w1, b1, w2, b2, batch2)


def _tc_head_body(s0_ref, c0_ref, s1_ref, c1_ref, s2_ref, c2_ref,
                  mw_ref, mb_ref, fw1_ref, fb1_ref, fw2_ref, fb2_ref,
                  o_ref):
    def hg(s_ref, c_ref):
        cnt = jnp.maximum(c_ref[...], 1.0)
        pooled = s_ref[...] / cnt.reshape(G, 1)
        return _mm(pooled, mw_ref[...]) + mb_ref[...]

    acc = (_mm(hg(s0_ref, c0_ref), fw1_ref[0:H, :])
           + _mm(hg(s1_ref, c1_ref), fw1_ref[H:2 * H, :])
           + _mm(hg(s2_ref, c2_ref), fw1_ref[2 * H:3 * H, :]))
    acc = jnp.maximum(acc + fb1_ref[...], 0.0)
    o_ref[...] = _mm(acc, fw2_ref[...]) + fb2_ref[...]


def _tc_head(s0, c0, s1, c1, s2, c2, mlp_W, mlp_b2, fW1, fb1_2, fW2, fb2_2):
    return pl.pallas_call(
        _tc_head_body,
        out_shape=jax.ShapeDtypeStruct((G, NC_OUT), jnp.float32),
    )(s0, c0, s1, c1, s2, c2, mlp_W, mlp_b2, fW1, fb1_2, fW2, fb2_2)


def kernel(x_org, edge_index_org, batch_org, x_c1, edge_index_c1, batch_c1,
           x_c2, edge_index_c2, batch_c2, conv_W1, conv_b1, conv_W2, conv_b2,
           mlp_W, mlp_b, final_W1, final_b1, final_W2, final_b2):
    zeros = jnp.zeros((ZROWS, D), jnp.float32)
    pad_src = jnp.zeros((EPAD - E,), jnp.int32)
    pad_dst = jnp.full((EPAD - E,), ACC_ROWS - 1, jnp.int32)

    b1r = conv_b1.reshape(3, 1, H)
    b2r = conv_b2.reshape(3, 1, H)

    def branch(x, ei, batch):
        src = jnp.concatenate([ei[0], pad_src]).reshape(NW, 2, HSTEPS, 1, CHUNK)
        dst = jnp.concatenate([ei[1], pad_dst]).reshape(NW, 2, HSTEPS, 1, CHUNK)
        idx = jnp.concatenate([src, dst], axis=3)
        batch2 = batch.reshape(N, 1)
        for l in range(2):
            parts = _sc_agg(x, idx, zeros)
            x = _tc_layer(x, parts, conv_W1[l], b1r[l], conv_W2[l], b2r[l])
        parts = _sc_agg(x, idx, zeros)
        return _tc_layer_pool(x, parts, conv_W1[2], b1r[2], conv_W2[2],
                              b2r[2], batch2)

    s0, c0 = branch(x_org, edge_index_org, batch_org)
    s1, c1 = branch(x_c1, edge_index_c1, batch_c1)
    s2, c2 = branch(x_c2, edge_index_c2, batch_c2)

    return _tc_head(s0, c0, s1, c1, s2, c2,
                    mlp_W, mlp_b.reshape(1, H),
                    final_W1, final_b1.reshape(1, H),
                    final_W2, final_b2.reshape(1, NC_OUT))
